# Initial kernel scaffold; baseline (speedup 1.0000x reference)
import jax
import jax.numpy as jnp
from jax.experimental import pallas as pl


def kernel(x, edge_index, W1, a1s, a1d, b1, W2, a2s, a2d, b2, W3, a3s, a3d, b3, fceW, fceb, fcdW, fcdb, W4, a4s, a4d, b4, W5, a5s, a5d, b5, W6, a6s, a6d, b6):
    raise NotImplementedError("write your pallas kernel here")


# speedup vs baseline: 23.4335x; 23.4335x over previous
import functools

import jax
import jax.numpy as jnp
from jax import lax
from jax.experimental import pallas as pl
from jax.experimental.pallas import tpu as pltpu
from jax.experimental.pallas import tpu_sc as plsc

N_NODES = 10000
NPAD = 10112
ROWS_PER_TILE = NPAD // 16
E_REAL = 320000
E_TOT = E_REAL + N_NODES
CHUNK = 128
N_WORKERS = 32
CHUNKS_PER_WORKER = -(-E_TOT // (CHUNK * N_WORKERS))
E_PAD = CHUNKS_PER_WORKER * CHUNK * N_WORKERS
EDGES_PER_WORKER = CHUNKS_PER_WORKER * CHUNK
TAIL = 16
EPS = 1e-16

_SC_PARAMS = pltpu.CompilerParams(
    needs_layout_passes=False, use_tc_tiling_on_sc=False)


def _expand_attn(a):
    h, c = a.shape
    eye = jnp.eye(h, dtype=a.dtype)
    return (eye[:, None, :] * a[:, :, None]).reshape(h * c, h)


@functools.cache
def _sc_gat(F, H):
    FT = F + TAIL
    cph = F // H
    mesh = plsc.VectorSubcoreMesh(core_axis_name="c", subcore_axis_name="s")

    def body(h_hbm, ad_hbm, edges_hbm, zeros_hbm, out_hbm,
             src_v, dst_v, rows_v, orows_v, ad_v, acc, sem, sem2):
        cid = lax.axis_index("c")
        sid = lax.axis_index("s")
        wid = cid * 16 + sid
        row0 = sid * ROWS_PER_TILE

        pltpu.sync_copy(zeros_hbm.at[pl.ds(row0, ROWS_PER_TILE)],
                        acc.at[pl.ds(row0, ROWS_PER_TILE)])

        def zb(j, _):
            orows_v[j, pl.ds(F, TAIL)] = jnp.zeros((16,), jnp.float32)
            return 0

        lax.fori_loop(0, CHUNK, zb, 0)
        plsc.subcore_barrier()

        base = wid * EDGES_PER_WORKER

        def chunk_body(ci, _):
            off = base + ci * CHUNK
            pltpu.sync_copy(edges_hbm.at[0, pl.ds(off, CHUNK)], src_v)
            pltpu.sync_copy(edges_hbm.at[1, pl.ds(off, CHUNK)], dst_v)
            cp1 = pltpu.async_copy(h_hbm.at[src_v], rows_v, sem)
            cp3 = pltpu.async_copy(ad_hbm.at[dst_v], ad_v, sem2)
            cp1.wait()
            cp3.wait()
            for g in range(CHUNK // 16):
                rows_idx = lax.iota(jnp.int32, 16) + (g * 16)
                for h in range(H):
                    a_s = plsc.load_gather(
                        rows_v, [rows_idx, jnp.full((16,), F + h, jnp.int32)])
                    a_d = plsc.load_gather(
                        ad_v, [rows_idx, jnp.full((16,), h, jnp.int32)])
                    e = a_s + a_d
                    e = jnp.where(e >= 0.0, e, e * jnp.float32(0.2))
                    ex = jnp.exp(e)
                    plsc.store_scatter(
                        orows_v, [rows_idx, jnp.full((16,), F + h, jnp.int32)], ex)
                    for t in range(cph):
                        col = jnp.full((16,), h * cph + t, jnp.int32)
                        vals = plsc.load_gather(rows_v, [rows_idx, col])
                        plsc.store_scatter(orows_v, [rows_idx, col], vals * ex)

            pltpu.sync_copy(orows_v, acc.at[dst_v], add=True)
            return 0

        lax.fori_loop(0, CHUNKS_PER_WORKER, chunk_body, 0)
        plsc.subcore_barrier()
        pltpu.sync_copy(acc.at[pl.ds(row0, ROWS_PER_TILE)],
                        out_hbm.at[cid, pl.ds(row0, ROWS_PER_TILE)])

    return pl.kernel(
        body,
        out_type=jax.ShapeDtypeStruct((2, NPAD, FT), jnp.float32),
        mesh=mesh,
        compiler_params=_SC_PARAMS,
        scratch_types=[
            pltpu.VMEM((CHUNK,), jnp.int32),
            pltpu.VMEM((CHUNK,), jnp.int32),
            pltpu.VMEM((CHUNK, FT), jnp.float32),
            pltpu.VMEM((CHUNK, FT), jnp.float32),
            pltpu.VMEM((CHUNK, TAIL), jnp.float32),
            pltpu.VMEM_SHARED((NPAD, FT), jnp.float32),
            pltpu.SemaphoreType.DMA,
            pltpu.SemaphoreType.DMA,
        ],
    )


def _dot(a, b):
    return jnp.dot(a, b, preferred_element_type=jnp.float32)


def _attn_outs(h, ab_ref, haug_ref, adt_ref):
    ab = _dot(h, ab_ref[...])
    haug_ref[...] = jnp.concatenate([h, ab[:, :TAIL]], axis=1)
    adt_ref[...] = ab[:, TAIL:]


def _tc_first(x_pad, W, AB):
    F = W.shape[1]

    def body(x_ref, w_ref, ab_ref, haug_ref, adt_ref):
        h = _dot(x_ref[...], w_ref[...])
        _attn_outs(h, ab_ref, haug_ref, adt_ref)

    return pl.pallas_call(
        body,
        out_shape=[jax.ShapeDtypeStruct((NPAD, F + TAIL), jnp.float32),
                   jax.ShapeDtypeStruct((NPAD, TAIL), jnp.float32)],
    )(x_pad, W, AB)


def _gat_norm(U, F, H, b):
    parts = []
    for h in range(H):
        c = F // H
        s = U[:, F + h:F + h + 1]
        parts.append(U[:, h * c:(h + 1) * c] / (s + EPS))
    g = parts[0] if H == 1 else jnp.concatenate(parts, axis=1)
    return g + b


def _tc_mid(P, b, W, AB, F, H):
    Fn = W.shape[1]

    def body(p_ref, b_ref, w_ref, ab_ref, haug_ref, adt_ref):
        U = p_ref[0] + p_ref[1]
        g = _gat_norm(U, F, H, b_ref[...])
        xn = jnp.maximum(g, 0.0)
        hn = _dot(xn, w_ref[...])
        _attn_outs(hn, ab_ref, haug_ref, adt_ref)

    return pl.pallas_call(
        body,
        out_shape=[jax.ShapeDtypeStruct((NPAD, Fn + TAIL), jnp.float32),
                   jax.ShapeDtypeStruct((NPAD, TAIL), jnp.float32)],
    )(P, b.reshape(1, F), W, AB)


def _tc_autoenc(P, b, fceW, fceb, fcdW, fcdb, W, AB):
    Fn = W.shape[1]
    slope = (1.0 / 8.0 + 1.0 / 3.0) / 2.0

    def body(p_ref, b_ref, ew_ref, eb_ref, dw_ref, db_ref, w_ref, ab_ref,
             haug_ref, adt_ref):
        U = p_ref[0] + p_ref[1]
        g = _gat_norm(U, 64, 1, b_ref[...])
        x3 = jnp.maximum(g, 0.0)
        enc = _dot(x3, ew_ref[...]) + eb_ref[...]
        dec = _dot(enc, dw_ref[...]) + db_ref[...]
        dec = jnp.where(dec >= 0.0, dec, dec * slope)
        hn = _dot(dec, w_ref[...])
        _attn_outs(hn, ab_ref, haug_ref, adt_ref)

    return pl.pallas_call(
        body,
        out_shape=[jax.ShapeDtypeStruct((NPAD, Fn + TAIL), jnp.float32),
                   jax.ShapeDtypeStruct((NPAD, TAIL), jnp.float32)],
    )(P, b.reshape(1, 64), fceW, fceb.reshape(1, -1), fcdW, fcdb.reshape(1, -1),
      W, AB)


def _tc_final(P, b):
    def body(p_ref, b_ref, y_ref):
        U = p_ref[0] + p_ref[1]
        g = _gat_norm(U, 128, 1, b_ref[...])
        y_ref[...] = jnp.tanh(g)

    return pl.pallas_call(
        body,
        out_shape=jax.ShapeDtypeStruct((NPAD, 128), jnp.float32),
    )(P, b.reshape(1, 128))


def kernel(x, edge_index, W1, a1s, a1d, b1, W2, a2s, a2d, b2, W3, a3s, a3d, b3,
           fceW, fceb, fcdW, fcdb, W4, a4s, a4d, b4, W5, a5s, a5d, b5,
           W6, a6s, a6d, b6):
    x_pad = jnp.pad(x, ((0, NPAD - N_NODES), (0, 0)))
    si = jnp.arange(N_NODES, dtype=edge_index.dtype)
    pad_ids = jnp.full((E_PAD - E_TOT,), N_NODES, dtype=edge_index.dtype)
    src = jnp.concatenate([edge_index[0], si, pad_ids])
    dst = jnp.concatenate([edge_index[1], si, pad_ids])
    edges = jnp.stack([src, dst])

    def ab_mat(a_s, a_d):
        As, Ad = _expand_attn(a_s), _expand_attn(a_d)
        f, hn = As.shape
        ab = jnp.zeros((f, 2 * TAIL), jnp.float32)
        return ab.at[:, :hn].set(As).at[:, TAIL:TAIL + hn].set(Ad)

    zeros = {f: jnp.zeros((NPAD, f + TAIL), jnp.float32) for f in (64, 32, 16, 128)}

    h1, ad1 = _tc_first(x_pad, W1, ab_mat(a1s, a1d))
    P1 = _sc_gat(64, 4)(h1, ad1, edges, zeros[64])
    h2, ad2 = _tc_mid(P1, b1, W2, ab_mat(a2s, a2d), 64, 4)
    P2 = _sc_gat(64, 2)(h2, ad2, edges, zeros[64])
    h3, ad3 = _tc_mid(P2, b2, W3, ab_mat(a3s, a3d), 64, 2)
    P3 = _sc_gat(64, 1)(h3, ad3, edges, zeros[64])
    h4, ad4 = _tc_autoenc(P3, b3, fceW, fceb, fcdW, fcdb, W4,
                          ab_mat(a4s, a4d))
    P4 = _sc_gat(32, 1)(h4, ad4, edges, zeros[32])
    h5, ad5 = _tc_mid(P4, b4, W5, ab_mat(a5s, a5d), 32, 1)
    P5 = _sc_gat(16, 1)(h5, ad5, edges, zeros[16])
    h6, ad6 = _tc_mid(P5, b5, W6, ab_mat(a6s, a6d), 16, 1)
    P6 = _sc_gat(128, 1)(h6, ad6, edges, zeros[128])
    y = _tc_final(P6, b6)
    return y[:N_NODES]
